# baseline (speedup 1.0000x reference)
import functools

import jax
import jax.numpy as jnp
from jax import lax
from jax.experimental import pallas as pl
from jax.experimental.pallas import tpu as pltpu
from jax.experimental.pallas import tpu_sc as plsc

M = 16384
J = 2
K = 20
C = 32
ROW = K * C

NC, NS, L = 2, 16, 16
NW = NC * NS
M_PER_TILE = M // NW
MB = M_PER_TILE // L
G_PER_TILE = M_PER_TILE // 128
MG = M // 128

_mesh = plsc.VectorSubcoreMesh(core_axis_name="c", subcore_axis_name="s")


@functools.partial(
    pl.kernel,
    mesh=_mesh,
    out_type=jax.ShapeDtypeStruct((K, MG, J, 128), jnp.float32),
    scratch_types=[
        pltpu.VMEM((J * ROW,), jnp.float32),
        pltpu.VMEM((G_PER_TILE, J, 128), jnp.int32),
        pltpu.VMEM((K, G_PER_TILE, J, 128), jnp.float32),
        pltpu.SemaphoreType.DMA,
        pltpu.SemaphoreType.DMA,
        [pltpu.SemaphoreType.DMA] * G_PER_TILE,
    ],
    compiler_params=pltpu.CompilerParams(needs_layout_passes=False),
)
def _sc_gather(slab_hbm, idx1_hbm, out_hbm, slab_v, idx1_v, outbuf_v,
               sem_slab, sem_idx, sems_out):
    wid = lax.axis_index("s") * NC + lax.axis_index("c")

    cp_slab = pltpu.async_copy(slab_hbm, slab_v, sem_slab)
    cp_idx = pltpu.async_copy(
        idx1_hbm.at[pl.ds(wid * G_PER_TILE, G_PER_TILE)], idx1_v, sem_idx)
    cp_idx.wait()
    cp_slab.wait()

    out_cps = []
    for grp in range(G_PER_TILE):
        @plsc.parallel_loop(grp * 8, (grp + 1) * 8, unroll=2)
        def body(b, grp=grp):
            lo = (b % 8) * L
            for j in range(J):
                g = idx1_v[grp, j, pl.ds(lo, L)]
                for k in range(K):
                    val = plsc.load_gather(slab_v, [g + (j * ROW + k * C)])
                    outbuf_v[k, grp, j, pl.ds(lo, L)] = val

        out_cps.append(pltpu.async_copy(
            outbuf_v.at[:, grp], out_hbm.at[:, wid * G_PER_TILE + grp],
            sems_out[grp]))
    for cp in out_cps:
        cp.wait()


def kernel(x, index1, index2):
    i2 = index2.astype(jnp.int32)
    slab = jnp.take(x, i2, axis=0, mode="clip").reshape(-1)
    i1 = index1.astype(jnp.int32).reshape(MG, 128, J).transpose(0, 2, 1)
    out4 = _sc_gather(slab, i1)
    return out4.transpose(1, 3, 2, 0).reshape(M, J, K)


# speedup vs baseline: 1.1535x; 1.1535x over previous
import functools

import jax
import jax.numpy as jnp
from jax import lax
from jax.experimental import pallas as pl
from jax.experimental.pallas import tpu as pltpu
from jax.experimental.pallas import tpu_sc as plsc

M = 16384
J = 2
K = 20
C = 32
ROW = K * C

NC, NS, L = 2, 16, 16
NW = NC * NS
M_PER_TILE = M // NW
MB = M_PER_TILE // L
G_PER_TILE = M_PER_TILE // 128
MG = M // 128

_mesh = plsc.VectorSubcoreMesh(core_axis_name="c", subcore_axis_name="s")


@functools.partial(
    pl.kernel,
    mesh=_mesh,
    out_type=jax.ShapeDtypeStruct((K, MG, J, 128), jnp.float32),
    scratch_types=[
        pltpu.VMEM((J * ROW,), jnp.float32),
        pltpu.VMEM((G_PER_TILE, J, 128), jnp.int32),
        pltpu.VMEM((K, G_PER_TILE, J, 128), jnp.float32),
        pltpu.SemaphoreType.DMA,
        pltpu.SemaphoreType.DMA,
    ],
    compiler_params=pltpu.CompilerParams(needs_layout_passes=False),
)
def _sc_gather(slab_hbm, idx1_hbm, out_hbm, slab_v, idx1_v, outbuf_v,
               sem_slab, sem_idx):
    wid = lax.axis_index("s") * NC + lax.axis_index("c")

    cp_slab = pltpu.async_copy(slab_hbm, slab_v, sem_slab)
    cp_idx = pltpu.async_copy(
        idx1_hbm.at[pl.ds(wid * G_PER_TILE, G_PER_TILE)], idx1_v, sem_idx)
    cp_idx.wait()
    cp_slab.wait()

    @plsc.parallel_loop(0, MB, unroll=2)
    def body(b):
        grp = b // 8
        lo = (b % 8) * L
        for j in range(J):
            g = idx1_v[grp, j, pl.ds(lo, L)]
            for k in range(K):
                val = plsc.load_gather(slab_v, [g + (j * ROW + k * C)])
                outbuf_v[k, grp, j, pl.ds(lo, L)] = val

    pltpu.sync_copy(outbuf_v, out_hbm.at[:, pl.ds(wid * G_PER_TILE, G_PER_TILE)])


def kernel(x, index1, index2):
    i2 = index2.astype(jnp.int32)
    slab = jnp.take(x, i2, axis=0, mode="clip").reshape(-1)
    i1 = index1.astype(jnp.int32).reshape(MG, 128, J).transpose(0, 2, 1)
    out4 = _sc_gather(slab, i1)
    return out4.transpose(1, 3, 2, 0).reshape(M, J, K)
